# baseline (speedup 1.0000x reference)
import jax
import jax.numpy as jnp
from jax.experimental import pallas as pl
from jax.experimental.pallas import tpu as pltpu

_B, _N, _F_ATOM, _F_PHYS = 1024, 100, 38, 3
_F_TOT = _F_ATOM + _F_PHYS
_BB = 256
_G = _B // _BB
_CB = 512


def _sum_kernel(x_hbm, s_ref, b0, b1, s0, s1):
    bufs, sems = (b0, b1), (s0, s1)

    def start(i):
        return pltpu.make_async_copy(
            x_hbm.at[pl.ds(i * _CB, _CB)], bufs[i], sems[i])

    start(0).start()
    start(1).start()
    for i in range(2):
        start(i).wait()
        x = bufs[i][...]
        s_ref[pl.ds(i * _CB, _CB), :_F_TOT] = jnp.sum(x, axis=1)
        s_ref[pl.ds(i * _CB, _CB), 48:48 + _F_PHYS] = x[:, 0, _F_ATOM:]


def _head_kernel(s_ref, Wr_ref, br_ref, Wc_ref, bc_ref, W1_ref, b1_ref,
                 W5_ref, b5_ref, W6_ref, b6_ref, W7h_ref, W7p_ref, b7_ref,
                 out_ref):
    s = s_ref[...]
    xb = s[:, :_F_ATOM] * (1.0 / _N)
    phys = s[:, 48:48 + _F_PHYS]
    h = jax.nn.relu(jnp.dot(xb, Wr_ref[...], preferred_element_type=jnp.float32)
                    + br_ref[...])
    g = jax.nn.relu(jnp.dot(h, Wc_ref[...], preferred_element_type=jnp.float32)
                    + bc_ref[...]) * float(_N)
    d1 = jax.nn.relu(jnp.dot(g, W1_ref[...], preferred_element_type=jnp.float32)
                     + b1_ref[...])
    d5 = jnp.dot(d1, W5_ref[...], preferred_element_type=jnp.float32) + b5_ref[...]
    mv = jnp.dot(d5, W6_ref[...], preferred_element_type=jnp.float32) + b6_ref[...]
    out = mv * W7h_ref[0, 0] + jnp.dot(phys, W7p_ref[...],
                                       preferred_element_type=jnp.float32)
    out_ref[...] = out + b7_ref[...]


def kernel(inputs, W_rule, b_rule, W_conv, b_conv, W1, b1, W5, b5, W6, b6,
           W7, b7):
    B = inputs.shape[0]
    R = W_rule.shape[1]

    sums = pl.pallas_call(
        _sum_kernel,
        grid=(1,),
        in_specs=[pl.BlockSpec(memory_space=pl.ANY)],
        out_specs=pl.BlockSpec((_B, 64), lambda i: (0, 0)),
        out_shape=jax.ShapeDtypeStruct((_B, 64), jnp.float32),
        scratch_shapes=[
            pltpu.VMEM((_CB, _N, _F_TOT), jnp.float32),
            pltpu.VMEM((_CB, _N, _F_TOT), jnp.float32),
            pltpu.SemaphoreType.DMA,
            pltpu.SemaphoreType.DMA,
        ],
        compiler_params=pltpu.CompilerParams(
            vmem_limit_bytes=100_000_000),
    )(inputs)

    out = pl.pallas_call(
        _head_kernel,
        grid=(1,),
        in_specs=[
            pl.BlockSpec((_B, 64), lambda i: (0, 0)),
            pl.BlockSpec(W_rule.shape, lambda i: (0, 0)),
            pl.BlockSpec((1, R), lambda i: (0, 0)),
            pl.BlockSpec(W_conv.shape, lambda i: (0, 0)),
            pl.BlockSpec((1, W_conv.shape[1]), lambda i: (0, 0)),
            pl.BlockSpec(W1.shape, lambda i: (0, 0)),
            pl.BlockSpec((1, W1.shape[1]), lambda i: (0, 0)),
            pl.BlockSpec(W5.shape, lambda i: (0, 0)),
            pl.BlockSpec((1, W5.shape[1]), lambda i: (0, 0)),
            pl.BlockSpec(W6.shape, lambda i: (0, 0)),
            pl.BlockSpec((1, 1), lambda i: (0, 0)),
            pl.BlockSpec((1, 1), lambda i: (0, 0)),
            pl.BlockSpec((_F_PHYS, 1), lambda i: (0, 0)),
            pl.BlockSpec((1, 1), lambda i: (0, 0)),
        ],
        out_specs=pl.BlockSpec((B, 1), lambda i: (0, 0)),
        out_shape=jax.ShapeDtypeStruct((B, 1), jnp.float32),
    )(sums, W_rule, b_rule.reshape(1, -1), W_conv, b_conv.reshape(1, -1),
      W1, b1.reshape(1, -1), W5, b5.reshape(1, -1), W6, b6.reshape(1, -1),
      W7[0:1, :], W7[1:4, :], b7.reshape(1, -1))
    return out


# speedup vs baseline: 1.0636x; 1.0636x over previous
import jax
import jax.numpy as jnp
from jax.experimental import pallas as pl
from jax.experimental.pallas import tpu as pltpu

_B, _N, _F_ATOM, _F_PHYS = 1024, 100, 38, 3
_F_TOT = _F_ATOM + _F_PHYS
_BB = 256
_G = _B // _BB


def _sum_kernel(x_ref, s_ref):
    x = x_ref[...]
    s_ref[:, :_F_TOT] = jnp.sum(x, axis=1)
    s_ref[:, 48:48 + _F_PHYS] = x[:, 0, _F_ATOM:]


def _head_kernel(s_ref, Wr_ref, br_ref, Wc_ref, bc_ref, W1_ref, b1_ref,
                 W5_ref, b5_ref, W6_ref, b6_ref, W7h_ref, W7p_ref, b7_ref,
                 out_ref):
    s = s_ref[...]
    xb = s[:, :_F_ATOM] * (1.0 / _N)
    phys = s[:, 48:48 + _F_PHYS]
    h = jax.nn.relu(jnp.dot(xb, Wr_ref[...], preferred_element_type=jnp.float32)
                    + br_ref[...])
    g = jax.nn.relu(jnp.dot(h, Wc_ref[...], preferred_element_type=jnp.float32)
                    + bc_ref[...]) * float(_N)
    d1 = jax.nn.relu(jnp.dot(g, W1_ref[...], preferred_element_type=jnp.float32)
                     + b1_ref[...])
    d5 = jnp.dot(d1, W5_ref[...], preferred_element_type=jnp.float32) + b5_ref[...]
    mv = jnp.dot(d5, W6_ref[...], preferred_element_type=jnp.float32) + b6_ref[...]
    out = mv * W7h_ref[0, 0] + jnp.dot(phys, W7p_ref[...],
                                       preferred_element_type=jnp.float32)
    out_ref[...] = out + b7_ref[...]


def kernel(inputs, W_rule, b_rule, W_conv, b_conv, W1, b1, W5, b5, W6, b6,
           W7, b7):
    B = inputs.shape[0]
    R = W_rule.shape[1]

    sums = pl.pallas_call(
        _sum_kernel,
        grid=(_G,),
        in_specs=[pl.BlockSpec((_BB, _N, _F_TOT), lambda i: (i, 0, 0))],
        out_specs=pl.BlockSpec((_BB, 64), lambda i: (i, 0)),
        out_shape=jax.ShapeDtypeStruct((_B, 64), jnp.float32),
        compiler_params=pltpu.CompilerParams(
            dimension_semantics=(pltpu.PARALLEL,)),
    )(inputs)

    out = pl.pallas_call(
        _head_kernel,
        grid=(1,),
        in_specs=[
            pl.BlockSpec((_B, 64), lambda i: (0, 0)),
            pl.BlockSpec(W_rule.shape, lambda i: (0, 0)),
            pl.BlockSpec((1, R), lambda i: (0, 0)),
            pl.BlockSpec(W_conv.shape, lambda i: (0, 0)),
            pl.BlockSpec((1, W_conv.shape[1]), lambda i: (0, 0)),
            pl.BlockSpec(W1.shape, lambda i: (0, 0)),
            pl.BlockSpec((1, W1.shape[1]), lambda i: (0, 0)),
            pl.BlockSpec(W5.shape, lambda i: (0, 0)),
            pl.BlockSpec((1, W5.shape[1]), lambda i: (0, 0)),
            pl.BlockSpec(W6.shape, lambda i: (0, 0)),
            pl.BlockSpec((1, 1), lambda i: (0, 0)),
            pl.BlockSpec((1, 1), lambda i: (0, 0)),
            pl.BlockSpec((_F_PHYS, 1), lambda i: (0, 0)),
            pl.BlockSpec((1, 1), lambda i: (0, 0)),
        ],
        out_specs=pl.BlockSpec((B, 1), lambda i: (0, 0)),
        out_shape=jax.ShapeDtypeStruct((B, 1), jnp.float32),
    )(sums, W_rule, b_rule.reshape(1, -1), W_conv, b_conv.reshape(1, -1),
      W1, b1.reshape(1, -1), W5, b5.reshape(1, -1), W6, b6.reshape(1, -1),
      W7[0:1, :], W7[1:4, :], b7.reshape(1, -1))
    return out
